# baseline (speedup 1.0000x reference)
import functools

import jax
import jax.numpy as jnp
from jax import lax
from jax.experimental import pallas as pl
from jax.experimental.pallas import tpu as pltpu
from jax.experimental.pallas import tpu_sc as plsc

_EPS = 1e-12

_NC = 2
_NS = 16
_NW = _NC * _NS

_K = 2
_T = 80
_BS = 16


def _sc_gather_sum(dna, ideas, idxd3, idxi3, n_tokens, chunks, h, t):
    mesh = plsc.VectorSubcoreMesh(core_axis_name="c", subcore_axis_name="s")

    @functools.partial(
        pl.kernel,
        mesh=mesh,
        out_type=jax.ShapeDtypeStruct((n_tokens, h), jnp.float32),
        scratch_types=[
            pltpu.VMEM((chunks, t), jnp.int32),
            pltpu.VMEM((chunks, t), jnp.int32),
            pltpu.VMEM((t, h), jnp.float32),
            pltpu.VMEM((t, h), jnp.float32),
            pltpu.VMEM((t, h), jnp.float32),
            pltpu.VMEM((t, h), jnp.float32),
            pltpu.SemaphoreType.DMA,
            pltpu.SemaphoreType.DMA,
            pltpu.SemaphoreType.DMA,
            pltpu.SemaphoreType.DMA,
        ],
    )
    def k(dna_h, ideas_h, idxd_h, idxi_h, out_h, idxd_v, idxi_v,
          rows_d0, rows_i0, rows_d1, rows_i1, semd0, semi0, semd1, semi1):
        wid = lax.axis_index("s") * _NC + lax.axis_index("c")
        pltpu.sync_copy(idxd_h.at[wid], idxd_v)
        pltpu.sync_copy(idxi_h.at[wid], idxi_v)

        bufs = ((rows_d0, rows_i0, semd0, semi0),
                (rows_d1, rows_i1, semd1, semi1))

        def start(c, b):
            rows_d, rows_i, semd, semi = bufs[b]
            pltpu.async_copy(dna_h.at[idxd_v.at[c]], rows_d, semd)
            pltpu.async_copy(ideas_h.at[idxi_v.at[c]], rows_i, semi)

        def finish(c, b):
            rows_d, rows_i, semd, semi = bufs[b]
            pltpu.make_async_copy(dna_h.at[idxd_v.at[c]], rows_d, semd).wait()
            pltpu.make_async_copy(ideas_h.at[idxi_v.at[c]], rows_i, semi).wait()

            def tok(i, carry2):
                for u in range(2):
                    for j in range(h // 16):
                        sl = pl.ds(j * 16, 16)
                        rows_d[2 * i + u, sl] = (rows_d[2 * i + u, sl]
                                                 + rows_i[2 * i + u, sl])
                return carry2

            lax.fori_loop(0, t // 2, tok, 0)
            base = (wid * chunks + c) * t
            pltpu.sync_copy(rows_d, out_h.at[pl.ds(base, t)])

        start(0, 0)

        def pair(g, carry):
            for b in range(2):
                c = 2 * g + b

                @pl.when(c + 1 < chunks)
                def _():
                    start(c + 1, 1 - b)

                finish(c, b)
            return carry

        lax.fori_loop(0, chunks // 2, pair, 0)

    return k(dna, ideas, idxd3, idxi3)


def _tc_ln_body(steps, *refs):
    sums_refs = refs[:_K]
    idd_ref, idi_ref, pos_ref, wd0_ref, wi0_ref, g_ref, b_ref, o_ref = refs[_K:]
    kk = pl.program_id(0)

    def compute(x_ref):
        x = x_ref[...]
        md = (idd_ref[...] == 0).astype(jnp.float32)[..., None]
        mi = (idi_ref[...] == 0).astype(jnp.float32)[..., None]
        x = (x
             - md * wd0_ref[0][None, None, :]
             - mi * wi0_ref[0][None, None, :]
             + pos_ref[...][None, :, :])
        mean = jnp.mean(x, axis=-1, keepdims=True)
        xc = x - mean
        var = jnp.mean(xc * xc, axis=-1, keepdims=True)
        o_ref[...] = (xc * lax.rsqrt(var + _EPS) * g_ref[0][None, None, :]
                      + b_ref[0][None, None, :])

    for k in range(_K):
        @pl.when(kk == k)
        def _(k=k):
            compute(sums_refs[k])


def _tc_layernorm(sums_slices, ids_d, ids_i, pos, wd0, wi0, gamma2, beta2):
    b, s = ids_d.shape
    h = pos.shape[-1]
    bslice = b // _K
    steps = bslice // _BS

    def slice_spec(k):
        def imap(kk, i):
            return (jnp.where(kk == k, i, 0), 0, 0)
        return pl.BlockSpec((_BS, s, h), imap)

    in_specs = [slice_spec(k) for k in range(_K)]
    in_specs += [
        pl.BlockSpec((_BS, s), lambda kk, i: (kk * steps + i, 0)),
        pl.BlockSpec((_BS, s), lambda kk, i: (kk * steps + i, 0)),
        pl.BlockSpec((s, h), lambda kk, i: (0, 0)),
        pl.BlockSpec((1, h), lambda kk, i: (0, 0)),
        pl.BlockSpec((1, h), lambda kk, i: (0, 0)),
        pl.BlockSpec((1, h), lambda kk, i: (0, 0)),
        pl.BlockSpec((1, h), lambda kk, i: (0, 0)),
    ]
    sums3 = [x.reshape(bslice, s, h) for x in sums_slices]
    return pl.pallas_call(
        functools.partial(_tc_ln_body, steps),
        grid=(_K, steps),
        in_specs=in_specs,
        out_specs=pl.BlockSpec((_BS, s, h), lambda kk, i: (kk * steps + i, 0, 0)),
        out_shape=jax.ShapeDtypeStruct((b, s, h), jnp.float32),
    )(*sums3, ids_d, ids_i, pos, wd0, wi0, gamma2, beta2)


def kernel(input_ids_dna, input_ids_ideas, W_dna, W_ideas, W_pos, gamma, beta):
    b, s = input_ids_dna.shape
    v, h = W_dna.shape
    n_tokens = b * s
    n_slice = n_tokens // _K
    chunks = n_slice // (_NW * _T)

    idd_flat = input_ids_dna.reshape(n_tokens)
    idi_flat = input_ids_ideas.reshape(n_tokens)

    sums_slices = []
    for k in range(_K):
        idxd3 = lax.slice(idd_flat, (k * n_slice,), ((k + 1) * n_slice,))
        idxi3 = lax.slice(idi_flat, (k * n_slice,), ((k + 1) * n_slice,))
        sums_slices.append(_sc_gather_sum(
            W_dna, W_ideas,
            idxd3.reshape(_NW, chunks, _T), idxi3.reshape(_NW, chunks, _T),
            n_slice, chunks, h, _T))

    return _tc_layernorm(
        sums_slices,
        input_ids_dna,
        input_ids_ideas,
        W_pos[:s],
        W_dna[0:1],
        W_ideas[0:1],
        gamma.reshape(1, h),
        beta.reshape(1, h),
    )


# speedup vs baseline: 1.1919x; 1.1919x over previous
import functools

import jax
import jax.numpy as jnp
from jax import lax
from jax.experimental import pallas as pl
from jax.experimental.pallas import tpu as pltpu
from jax.experimental.pallas import tpu_sc as plsc

_EPS = 1e-12

_NC = 2
_NS = 16
_NW = _NC * _NS

_K = 4
_T = 80
_BS = 16


def _sc_gather_sum(dna, ideas, idxd3, idxi3, n_tokens, chunks, h, t):
    mesh = plsc.VectorSubcoreMesh(core_axis_name="c", subcore_axis_name="s")

    @functools.partial(
        pl.kernel,
        mesh=mesh,
        out_type=jax.ShapeDtypeStruct((n_tokens, h), jnp.float32),
        scratch_types=[
            pltpu.VMEM((chunks, t), jnp.int32),
            pltpu.VMEM((chunks, t), jnp.int32),
            pltpu.VMEM((t, h), jnp.float32),
            pltpu.VMEM((t, h), jnp.float32),
            pltpu.VMEM((t, h), jnp.float32),
            pltpu.VMEM((t, h), jnp.float32),
            pltpu.SemaphoreType.DMA,
            pltpu.SemaphoreType.DMA,
            pltpu.SemaphoreType.DMA,
            pltpu.SemaphoreType.DMA,
        ],
    )
    def k(dna_h, ideas_h, idxd_h, idxi_h, out_h, idxd_v, idxi_v,
          rows_d0, rows_i0, rows_d1, rows_i1, semd0, semi0, semd1, semi1):
        wid = lax.axis_index("s") * _NC + lax.axis_index("c")
        pltpu.sync_copy(idxd_h.at[wid], idxd_v)
        pltpu.sync_copy(idxi_h.at[wid], idxi_v)

        bufs = ((rows_d0, rows_i0, semd0, semi0),
                (rows_d1, rows_i1, semd1, semi1))

        def start(c, b):
            rows_d, rows_i, semd, semi = bufs[b]
            pltpu.async_copy(dna_h.at[idxd_v.at[c]], rows_d, semd)
            pltpu.async_copy(ideas_h.at[idxi_v.at[c]], rows_i, semi)

        def finish(c, b):
            rows_d, rows_i, semd, semi = bufs[b]
            pltpu.make_async_copy(dna_h.at[idxd_v.at[c]], rows_d, semd).wait()
            pltpu.make_async_copy(ideas_h.at[idxi_v.at[c]], rows_i, semi).wait()

            def tok(i, carry2):
                for u in range(2):
                    for j in range(h // 16):
                        sl = pl.ds(j * 16, 16)
                        rows_d[2 * i + u, sl] = (rows_d[2 * i + u, sl]
                                                 + rows_i[2 * i + u, sl])
                return carry2

            lax.fori_loop(0, t // 2, tok, 0)
            base = (wid * chunks + c) * t
            pltpu.sync_copy(rows_d, out_h.at[pl.ds(base, t)])

        start(0, 0)

        def pair(g, carry):
            for b in range(2):
                c = 2 * g + b

                @pl.when(c + 1 < chunks)
                def _():
                    start(c + 1, 1 - b)

                finish(c, b)
            return carry

        lax.fori_loop(0, chunks // 2, pair, 0)

    return k(dna, ideas, idxd3, idxi3)


def _tc_ln_body(has_alias, x_ref, idd_ref, idi_ref, pos_ref, wd0_ref, wi0_ref,
                g_ref, b_ref, *rest):
    o_ref = rest[-1]
    x = x_ref[...]
    md = (idd_ref[...] == 0).astype(jnp.float32)[..., None]
    mi = (idi_ref[...] == 0).astype(jnp.float32)[..., None]
    x = (x
         - md * wd0_ref[0][None, None, :]
         - mi * wi0_ref[0][None, None, :]
         + pos_ref[...][None, :, :])
    mean = jnp.mean(x, axis=-1, keepdims=True)
    xc = x - mean
    var = jnp.mean(xc * xc, axis=-1, keepdims=True)
    o_ref[...] = (xc * lax.rsqrt(var + _EPS) * g_ref[0][None, None, :]
                  + b_ref[0][None, None, :])


def _tc_ln_slice(k, sums_k, ids_d, ids_i, pos, wd0, wi0, gamma2, beta2,
                 prev_buf):
    b, s = ids_d.shape
    h = pos.shape[-1]
    bslice = b // _K
    steps = bslice // _BS
    sums3 = sums_k.reshape(bslice, s, h)

    in_specs = [
        pl.BlockSpec((_BS, s, h), lambda i: (i, 0, 0)),
        pl.BlockSpec((_BS, s), lambda i: (k * steps + i, 0)),
        pl.BlockSpec((_BS, s), lambda i: (k * steps + i, 0)),
        pl.BlockSpec((s, h), lambda i: (0, 0)),
        pl.BlockSpec((1, h), lambda i: (0, 0)),
        pl.BlockSpec((1, h), lambda i: (0, 0)),
        pl.BlockSpec((1, h), lambda i: (0, 0)),
        pl.BlockSpec((1, h), lambda i: (0, 0)),
    ]
    args = [sums3, ids_d, ids_i, pos, wd0, wi0, gamma2, beta2]
    aliases = {}
    if prev_buf is not None:
        in_specs.append(pl.BlockSpec(memory_space=pl.ANY))
        args.append(prev_buf)
        aliases = {8: 0}
    return pl.pallas_call(
        functools.partial(_tc_ln_body, prev_buf is not None),
        grid=(steps,),
        in_specs=in_specs,
        out_specs=pl.BlockSpec((_BS, s, h), lambda i: (k * steps + i, 0, 0)),
        out_shape=jax.ShapeDtypeStruct((b, s, h), jnp.float32),
        input_output_aliases=aliases,
    )(*args)


def kernel(input_ids_dna, input_ids_ideas, W_dna, W_ideas, W_pos, gamma, beta):
    b, s = input_ids_dna.shape
    v, h = W_dna.shape
    n_tokens = b * s
    n_slice = n_tokens // _K
    chunks = n_slice // (_NW * _T)

    idd_flat = input_ids_dna.reshape(n_tokens)
    idi_flat = input_ids_ideas.reshape(n_tokens)

    pos = W_pos[:s]
    wd0 = W_dna[0:1]
    wi0 = W_ideas[0:1]
    gamma2 = gamma.reshape(1, h)
    beta2 = beta.reshape(1, h)

    buf = None
    for k in range(_K):
        idxd = lax.slice(idd_flat, (k * n_slice,), ((k + 1) * n_slice,))
        idxi = lax.slice(idi_flat, (k * n_slice,), ((k + 1) * n_slice,))
        sums_k = _sc_gather_sum(
            W_dna, W_ideas,
            idxd.reshape(_NW, chunks, _T), idxi.reshape(_NW, chunks, _T),
            n_slice, chunks, h, _T)
        buf = _tc_ln_slice(k, sums_k, input_ids_dna, input_ids_ideas,
                           pos, wd0, wi0, gamma2, beta2, buf)
    return buf
